# baseline (speedup 1.0000x reference)
import functools

import jax
import jax.numpy as jnp
from jax import lax
from jax.experimental import pallas as pl
from jax.experimental.pallas import tpu as pltpu
from jax.experimental.pallas import tpu_sc as plsc

_B = 16384
_D = 64
_V = 1000000
_NC = 2
_NS = 16
_NW = _NC * _NS
_BPW = _B // _NW

_TW = 24576
_NPB = -(-_V // (2 * _TW))
_LASTB = (_V - 1) // _TW
_PR = _NPB * _TW


def _tc_pack_rows(tab_t):
  def body(t1_ref, t2_ref, o_ref):
    eye = (lax.broadcasted_iota(jnp.int32, (_D, _D), 0)
           == lax.broadcasted_iota(jnp.int32, (_D, _D), 1)
           ).astype(jnp.bfloat16)

    def tr(ref):
      return lax.dot_general(ref[...].astype(jnp.bfloat16), eye,
                             (((0,), (0,)), ((), ())),
                             preferred_element_type=jnp.float32)

    o_ref[...] = jnp.concatenate(
        [tr(t1_ref), tr(t2_ref)], axis=1).astype(jnp.bfloat16)

  mesh = pltpu.create_tensorcore_mesh("core", num_cores=2)

  @functools.partial(
      pl.kernel,
      mesh=mesh,
      out_type=jax.ShapeDtypeStruct((_PR, 2 * _D), jnp.bfloat16),
  )
  def k(t_hbm, o_hbm):
    pltpu.emit_pipeline(
        body,
        grid=(_NPB,),
        in_specs=[pl.BlockSpec((_D, _TW), lambda i: (0, 2 * i)),
                  pl.BlockSpec((_D, _TW),
                               lambda i: (0, jnp.minimum(2 * i + 1, _LASTB)))],
        out_specs=[pl.BlockSpec((_TW, 2 * _D), lambda i: (i, 0))],
        core_axis_name="core",
        dimension_semantics=(pltpu.PARALLEL,),
    )(t_hbm, t_hbm, o_hbm)

  return k(tab_t)


def _pair_idx(r):
  return ((r // (2 * _TW)) * _TW + (r % _TW)) // 2


def _slot(r):
  return 2 * (r % 2) + (r // _TW) % 2


def _sc_gather(packed, idx_list):
  n = len(idx_list)
  mesh = plsc.VectorSubcoreMesh(core_axis_name="c", subcore_axis_name="s")

  @functools.partial(
      pl.kernel,
      mesh=mesh,
      out_type=[jax.ShapeDtypeStruct((_B, 2 * _D), jnp.int32)] * n,
      scratch_types=(
          [pltpu.VMEM((_BPW,), jnp.int32)] * n
          + [pltpu.VMEM((_BPW, 2 * _D), jnp.int32)] * n
          + [pltpu.SemaphoreType.DMA] * n
      ),
  )
  def k(tab, *refs):
    idx_hbm = refs[:n]
    outs = refs[n:2 * n]
    idx_v = refs[2 * n:3 * n]
    rows_v = refs[3 * n:4 * n]
    sems = refs[4 * n:]
    tab32 = tab.bitcast(jnp.int32)
    wid = lax.axis_index("s") * _NC + lax.axis_index("c")
    base = wid * _BPW
    copies = []
    for t in range(n):
      pltpu.sync_copy(idx_hbm[t].at[pl.ds(base, _BPW)], idx_v[t])
      copies.append(pltpu.async_copy(tab32.at[idx_v[t]], rows_v[t], sems[t]))
    for t in range(n):
      copies[t].wait()
      pltpu.sync_copy(rows_v[t], outs[t].at[pl.ds(base, _BPW)])

  return k(packed, *idx_list)


_TB = 2048


def _tc_dots(u2, ei2, ej2, us, isl, js):
  def body(u_ref, ei_ref, ej_ref, us_ref, is_ref, js_ref, pi_ref, pj_ref):
    def sel(rows_ref, s_ref):
      s = s_ref[0, :][:, None]
      x = rows_ref[...]
      a = jnp.where(s % 2 == 1, x[:, _D:], x[:, :_D])
      lo = lax.bitcast_convert_type(a << 16, jnp.float32)
      hi = lax.bitcast_convert_type(a & jnp.int32(-65536), jnp.float32)
      return jnp.where(s >= 2, hi, lo)

    uu = sel(u_ref, us_ref)
    pi_ref[...] = jnp.sum(uu * sel(ei_ref, is_ref), axis=1)[None, :]
    pj_ref[...] = jnp.sum(uu * sel(ej_ref, js_ref), axis=1)[None, :]

  mesh = pltpu.create_tensorcore_mesh("core", num_cores=2)

  @functools.partial(
      pl.kernel,
      mesh=mesh,
      out_type=[jax.ShapeDtypeStruct((1, _B), jnp.float32)] * 2,
  )
  def k(u_hbm, ei_hbm, ej_hbm, us_hbm, is_hbm, js_hbm, pi_hbm, pj_hbm):
    pltpu.emit_pipeline(
        body,
        grid=(_B // _TB,),
        in_specs=(
            [pl.BlockSpec((_TB, 2 * _D), lambda i: (i, 0))] * 3
            + [pl.BlockSpec((1, _TB), lambda i: (0, i))] * 3
        ),
        out_specs=[pl.BlockSpec((1, _TB), lambda i: (0, i))] * 2,
        core_axis_name="core",
        dimension_semantics=(pltpu.PARALLEL,),
    )(u_hbm, ei_hbm, ej_hbm, us_hbm, is_hbm, js_hbm, pi_hbm, pj_hbm)

  return k(u2, ei2, ej2, us, isl, js)


def kernel(user, item_i, item_j, distance_ij, embed_user_w, embed_item_w):
  uidx = user.astype(jnp.int32)
  iidx = item_i.astype(jnp.int32)
  jidx = item_j.astype(jnp.int32)
  packed_u = _tc_pack_rows(embed_user_w.T)
  (u2,) = _sc_gather(packed_u, [_pair_idx(uidx)])
  packed_i = _tc_pack_rows(embed_item_w.T)
  (ei2,) = _sc_gather(packed_i, [_pair_idx(iidx)])
  (ej2,) = _sc_gather(packed_i, [_pair_idx(jidx)])
  pi, pj = _tc_dots(u2, ei2, ej2, _slot(uidx)[None, :],
                    _slot(iidx)[None, :], _slot(jidx)[None, :])
  return (pi.reshape(_B), pj.reshape(_B), distance_ij)


# speedup vs baseline: 1.0418x; 1.0418x over previous
import functools

import jax
import jax.numpy as jnp
from jax import lax
from jax.experimental import pallas as pl
from jax.experimental.pallas import tpu as pltpu
from jax.experimental.pallas import tpu_sc as plsc

_B = 16384
_D = 64
_V = 1000000
_NC = 2
_NS = 16
_NW = _NC * _NS
_BPW = _B // _NW

_TW = 16384
_NPB = -(-_V // (2 * _TW))
_LASTB = (_V - 1) // _TW
_PR = _NPB * _TW


def _tc_pack_rows(tab_t):
  def body(t_ref, o_ref):
    eye = (lax.broadcasted_iota(jnp.int32, (_D, _D), 0)
           == lax.broadcasted_iota(jnp.int32, (_D, _D), 1)
           ).astype(jnp.bfloat16)
    x = t_ref[...].astype(jnp.bfloat16)

    def tr(h):
      return lax.dot_general(x[:, h * _TW:(h + 1) * _TW], eye,
                             (((0,), (0,)), ((), ())),
                             preferred_element_type=jnp.float32)

    o_ref[...] = jnp.concatenate([tr(0), tr(1)], axis=1).astype(jnp.bfloat16)

  mesh = pltpu.create_tensorcore_mesh("core", num_cores=2)

  @functools.partial(
      pl.kernel,
      mesh=mesh,
      out_type=jax.ShapeDtypeStruct((_PR, 2 * _D), jnp.bfloat16),
  )
  def k(t_hbm, o_hbm):
    pltpu.emit_pipeline(
        body,
        grid=(_NPB,),
        in_specs=[pl.BlockSpec((_D, 2 * _TW), lambda i: (0, i))],
        out_specs=[pl.BlockSpec((_TW, 2 * _D), lambda i: (i, 0))],
        core_axis_name="core",
        dimension_semantics=(pltpu.PARALLEL,),
    )(t_hbm, o_hbm)

  return k(tab_t)


def _pair_idx(r):
  return ((r // (2 * _TW)) * _TW + (r % _TW)) // 2


def _slot(r):
  return 2 * (r % 2) + (r // _TW) % 2


def _sc_gather(packed, idx_list):
  n = len(idx_list)
  mesh = plsc.VectorSubcoreMesh(core_axis_name="c", subcore_axis_name="s")

  @functools.partial(
      pl.kernel,
      mesh=mesh,
      out_type=[jax.ShapeDtypeStruct((_B, 2 * _D), jnp.int32)] * n,
      scratch_types=(
          [pltpu.VMEM((_BPW,), jnp.int32)] * n
          + [pltpu.VMEM((_BPW, 2 * _D), jnp.int32)] * n
          + [pltpu.SemaphoreType.DMA] * n
      ),
  )
  def k(tab, *refs):
    idx_hbm = refs[:n]
    outs = refs[n:2 * n]
    idx_v = refs[2 * n:3 * n]
    rows_v = refs[3 * n:4 * n]
    sems = refs[4 * n:]
    tab32 = tab.bitcast(jnp.int32)
    wid = lax.axis_index("s") * _NC + lax.axis_index("c")
    base = wid * _BPW
    copies = []
    for t in range(n):
      pltpu.sync_copy(idx_hbm[t].at[pl.ds(base, _BPW)], idx_v[t])
      copies.append(pltpu.async_copy(tab32.at[idx_v[t]], rows_v[t], sems[t]))
    for t in range(n):
      copies[t].wait()
      pltpu.sync_copy(rows_v[t], outs[t].at[pl.ds(base, _BPW)])

  return k(packed, *idx_list)


_TB = 2048


def _tc_dots(u2, ei2, ej2, us, isl, js):
  def body(u_ref, ei_ref, ej_ref, us_ref, is_ref, js_ref, pi_ref, pj_ref):
    def sel(rows_ref, s_ref):
      s = s_ref[0, :][:, None]
      x = rows_ref[...]
      a = jnp.where(s % 2 == 1, x[:, _D:], x[:, :_D])
      lo = lax.bitcast_convert_type(a << 16, jnp.float32)
      hi = lax.bitcast_convert_type(a & jnp.int32(-65536), jnp.float32)
      return jnp.where(s >= 2, hi, lo)

    uu = sel(u_ref, us_ref)
    ones = jnp.full((1, _D), 1, dtype=jnp.bfloat16)

    def rsum(prod):
      return lax.dot_general(ones, prod.astype(jnp.bfloat16),
                             (((1,), (1,)), ((), ())),
                             preferred_element_type=jnp.float32)

    pi_ref[...] = rsum(uu * sel(ei_ref, is_ref))
    pj_ref[...] = rsum(uu * sel(ej_ref, js_ref))

  mesh = pltpu.create_tensorcore_mesh("core", num_cores=2)

  @functools.partial(
      pl.kernel,
      mesh=mesh,
      out_type=[jax.ShapeDtypeStruct((1, _B), jnp.float32)] * 2,
  )
  def k(u_hbm, ei_hbm, ej_hbm, us_hbm, is_hbm, js_hbm, pi_hbm, pj_hbm):
    pltpu.emit_pipeline(
        body,
        grid=(_B // _TB,),
        in_specs=(
            [pl.BlockSpec((_TB, 2 * _D), lambda i: (i, 0))] * 3
            + [pl.BlockSpec((1, _TB), lambda i: (0, i))] * 3
        ),
        out_specs=[pl.BlockSpec((1, _TB), lambda i: (0, i))] * 2,
        core_axis_name="core",
        dimension_semantics=(pltpu.PARALLEL,),
    )(u_hbm, ei_hbm, ej_hbm, us_hbm, is_hbm, js_hbm, pi_hbm, pj_hbm)

  return k(u2, ei2, ej2, us, isl, js)


def kernel(user, item_i, item_j, distance_ij, embed_user_w, embed_item_w):
  uidx = user.astype(jnp.int32)
  iidx = item_i.astype(jnp.int32)
  jidx = item_j.astype(jnp.int32)
  packed_u = _tc_pack_rows(embed_user_w.T)
  (u2,) = _sc_gather(packed_u, [_pair_idx(uidx)])
  packed_i = _tc_pack_rows(embed_item_w.T)
  (ei2,) = _sc_gather(packed_i, [_pair_idx(iidx)])
  (ej2,) = _sc_gather(packed_i, [_pair_idx(jidx)])
  pi, pj = _tc_dots(u2, ei2, ej2, _slot(uidx)[None, :],
                    _slot(iidx)[None, :], _slot(jidx)[None, :])
  return (pi.reshape(_B), pj.reshape(_B), distance_ij)
